# speedup vs baseline: 1.1791x; 1.1791x over previous
import functools

import jax
import jax.numpy as jnp
from jax import lax
from jax.experimental import pallas as pl
from jax.experimental.pallas import tpu as pltpu
from jax.experimental.pallas import tpu_sc as plsc

BATCH = 16384
FEAT = 128
NUM_CORES = 2
NUM_SUBCORES = 16
NUM_WORKERS = NUM_CORES * NUM_SUBCORES
ROWS_PER_WORKER = BATCH // NUM_WORKERS
CHUNK = 128
NCHUNK = ROWS_PER_WORKER // CHUNK
LANES = 16

_mesh = plsc.VectorSubcoreMesh(core_axis_name="c", subcore_axis_name="s")


@functools.partial(
    pl.kernel,
    mesh=_mesh,
    out_type=jax.ShapeDtypeStruct((BATCH, FEAT), jnp.float32),
    scratch_types=[
        pltpu.VMEM((NCHUNK, CHUNK), jnp.int32),
        pltpu.VMEM((CHUNK, FEAT), jnp.float32),
        pltpu.VMEM((CHUNK, FEAT), jnp.float32),
        pltpu.VMEM((CHUNK, FEAT), jnp.float32),
        pltpu.SemaphoreType.DMA,
        pltpu.SemaphoreType.DMA,
        pltpu.SemaphoreType.DMA,
    ],
)
def _film_sc(x_hbm, ids_hbm, gamma_hbm, beta_hbm, out_hbm,
             idx_v, g_v, b_v, x_v, sem_g, sem_b, sem_x):
    wid = lax.axis_index("s") * NUM_CORES + lax.axis_index("c")
    base = wid * ROWS_PER_WORKER

    pltpu.sync_copy(ids_hbm.at[pl.ds(wid * NCHUNK, NCHUNK)], idx_v)

    for c in range(NCHUNK):
        off = base + c * CHUNK
        cg = pltpu.async_copy(gamma_hbm.at[idx_v.at[c]], g_v, sem_g)
        cb = pltpu.async_copy(beta_hbm.at[idx_v.at[c]], b_v, sem_b)
        cx = pltpu.async_copy(x_hbm.at[pl.ds(off, CHUNK)], x_v, sem_x)
        cg.wait()
        cb.wait()
        cx.wait()

        def row_body(r, carry):
            for j in range(FEAT // LANES):
                sl = pl.ds(j * LANES, LANES)
                g_v[r, sl] = g_v[r, sl] * x_v[r, sl] + b_v[r, sl]
            return carry

        lax.fori_loop(0, CHUNK, row_body, 0)
        pltpu.sync_copy(g_v, out_hbm.at[pl.ds(off, CHUNK)])


def kernel(x, domain_ids, gamma, beta):
    ids2d = domain_ids.astype(jnp.int32).reshape(NUM_WORKERS * NCHUNK, CHUNK)
    return _film_sc(x, ids2d, gamma, beta)


# speedup vs baseline: 1.4440x; 1.2246x over previous
import functools

import jax
import jax.numpy as jnp
from jax import lax
from jax.experimental import pallas as pl
from jax.experimental.pallas import tpu as pltpu
from jax.experimental.pallas import tpu_sc as plsc

BATCH = 16384
FEAT = 128
NUM_CORES = 2
NUM_SUBCORES = 16
NUM_WORKERS = NUM_CORES * NUM_SUBCORES
ROWS_PER_WORKER = BATCH // NUM_WORKERS
CHUNK = 128
NCHUNK = ROWS_PER_WORKER // CHUNK
LANES = 16

_mesh = plsc.VectorSubcoreMesh(core_axis_name="c", subcore_axis_name="s")


@functools.partial(
    pl.kernel,
    mesh=_mesh,
    out_type=jax.ShapeDtypeStruct((BATCH, FEAT), jnp.float32),
    scratch_types=[
        pltpu.VMEM((NCHUNK, CHUNK), jnp.int32),
        pltpu.VMEM((2, CHUNK, FEAT), jnp.float32),
        pltpu.VMEM((2, CHUNK, FEAT), jnp.float32),
        pltpu.VMEM((2, CHUNK, FEAT), jnp.float32),
        pltpu.SemaphoreType.DMA,
        pltpu.SemaphoreType.DMA,
        pltpu.SemaphoreType.DMA,
        pltpu.SemaphoreType.DMA,
    ],
)
def _film_sc(x_hbm, ids_hbm, gamma_hbm, beta_hbm, out_hbm,
             idx_v, g_v, b_v, x_v, sem_g, sem_b, sem_x, sem_o):
    wid = lax.axis_index("s") * NUM_CORES + lax.axis_index("c")
    base = wid * ROWS_PER_WORKER

    pltpu.sync_copy(ids_hbm.at[pl.ds(wid * NCHUNK, NCHUNK)], idx_v)

    def issue(c):
        s = c % 2
        off = base + c * CHUNK
        return (
            pltpu.async_copy(gamma_hbm.at[idx_v.at[c]], g_v.at[s], sem_g),
            pltpu.async_copy(beta_hbm.at[idx_v.at[c]], b_v.at[s], sem_b),
            pltpu.async_copy(x_hbm.at[pl.ds(off, CHUNK)], x_v.at[s], sem_x),
        )

    handles = issue(0)
    wb = None
    for c in range(NCHUNK):
        s = c % 2
        if c + 1 < NCHUNK:
            if wb is not None:
                wb.wait()
                wb = None
            nxt = issue(c + 1)
        for h in handles:
            h.wait()

        def row_body(r, carry):
            for j in range(FEAT // LANES):
                sl = pl.ds(j * LANES, LANES)
                b_v[s, r, sl] = g_v[s, r, sl] * x_v[s, r, sl] + b_v[s, r, sl]
            return carry

        lax.fori_loop(0, CHUNK, row_body, 0)
        if wb is not None:
            wb.wait()
        wb = pltpu.async_copy(b_v.at[s], out_hbm.at[pl.ds(base + c * CHUNK, CHUNK)], sem_o)
        if c + 1 < NCHUNK:
            handles = nxt
    wb.wait()


def kernel(x, domain_ids, gamma, beta):
    ids2d = domain_ids.astype(jnp.int32).reshape(NUM_WORKERS * NCHUNK, CHUNK)
    return _film_sc(x, ids2d, gamma, beta)
